# Initial kernel scaffold; baseline (speedup 1.0000x reference)
import jax
import jax.numpy as jnp
from jax.experimental import pallas as pl


def kernel(x, edge_index, edge_attr, batch, global_features, W1, as1, ad1, We1, ae1, b1, g1, be1, W2, as2, ad2, We2, ae2, b2, g2, be2, Wg, bg, Wi1, Wh1, bi1, bh1, Wi2, Wh2, bi2, bh2, Wo1, bo1, Wo2, bo2, Wb1, bb1, Wb2, bb2):
    raise NotImplementedError("write your pallas kernel here")


# speedup vs baseline: 5.7616x; 5.7616x over previous
import functools
import jax
import jax.numpy as jnp
from jax import lax
from jax.experimental import pallas as pl
from jax.experimental.pallas import tpu as pltpu
from jax.experimental.pallas import tpu_sc as plsc

N = 10000
E = 160000
F_IN = 256
EDGE_DIM = 16
HID = 128
LSTM_H = 256
GFEAT = 32
NG = 240
SEQ = 6
BS = NG // SEQ

NC = 2
NS = 16
CH = 128

N_PAD = 10112
E_PAD16 = 161792
E_PAD32 = 163840
NP_PAD32 = 12288
NG_PAD = 256


def _ceil_to(x, m):
    return (x + m - 1) // m * m


@functools.partial(jax.jit, static_argnames=("n_pad",))
def _sc_segsum(vals, idx, zeros, *, n_pad):
    M, W = vals.shape
    per_tile = M // (NC * NS)
    chunks = per_tile // CH
    rows_pt = n_pad // NS
    mesh = plsc.VectorSubcoreMesh(core_axis_name="c", subcore_axis_name="s")

    @functools.partial(
        pl.kernel, mesh=mesh,
        out_type=jax.ShapeDtypeStruct((NC * n_pad, W), jnp.float32),
        scratch_types=[
            pltpu.VMEM((CH,), jnp.int32),
            pltpu.VMEM((CH, W), jnp.float32),
            pltpu.VMEM_SHARED((n_pad, W), jnp.float32),
        ],
    )
    def k(vals_hbm, idx_hbm, zeros_hbm, out_hbm, idx_v, val_v, acc_sh):
        c = lax.axis_index("c")
        s = lax.axis_index("s")
        wid = s * NC + c

        if True:
            pltpu.sync_copy(zeros_hbm, acc_sh.at[pl.ds(s * rows_pt, rows_pt)])
            plsc.subcore_barrier()

            def body(i, carry):
                off = wid * per_tile + i * CH
                pltpu.sync_copy(idx_hbm.at[pl.ds(off, CH)], idx_v)
                pltpu.sync_copy(vals_hbm.at[pl.ds(off, CH)], val_v)
                pltpu.sync_copy(val_v, acc_sh.at[idx_v], add=True)
                return carry

            lax.fori_loop(0, chunks, body, 0)
            plsc.subcore_barrier()
            pltpu.sync_copy(acc_sh.at[pl.ds(s * rows_pt, rows_pt)],
                            out_hbm.at[pl.ds(c * n_pad + s * rows_pt, rows_pt)])

    return k(vals, idx, zeros).reshape(NC, n_pad, W)


def _sc_aggregate(table, gidx_flat, alpha_flat, dst, zeros, *, n_heads):
    if n_heads == 4:
        e_pad, heads_per_core, n_out_maj = E_PAD16, 2, 4
        per_tile = e_pad // NS
    else:
        e_pad, heads_per_core, n_out_maj = E_PAD32, 1, 2
        per_tile = e_pad // (NC * NS)
    chunks = per_tile // CH
    rows_pt = N_PAD // NS
    mesh = plsc.VectorSubcoreMesh(core_axis_name="c", subcore_axis_name="s")

    @functools.partial(
        pl.kernel, mesh=mesh,
        out_type=jax.ShapeDtypeStruct((n_out_maj * N_PAD, HID), jnp.float32),
        scratch_types=[
            pltpu.VMEM((CH,), jnp.int32),
            pltpu.VMEM((CH,), jnp.int32),
            pltpu.VMEM((CH,), jnp.float32),
            pltpu.VMEM((CH, HID), jnp.float32),
            pltpu.VMEM_SHARED((N_PAD, HID), jnp.float32),
            pltpu.SemaphoreType.DMA,
        ],
    )
    def k(table_hbm, gidx_hbm, alpha_hbm, dst_hbm, zeros_hbm, out_hbm,
          gidx_v, dst_v, alpha_v, rows_v, acc_sh, sem):
        c = lax.axis_index("c")
        s = lax.axis_index("s")

        if True:
            for klocal in range(heads_per_core):
                hd = c * heads_per_core + klocal
                pltpu.sync_copy(zeros_hbm,
                                acc_sh.at[pl.ds(s * rows_pt, rows_pt)])
                plsc.subcore_barrier()

                if n_heads == 4:
                    tile_base = s * per_tile
                else:
                    tile_base = (s * NC + c) * per_tile

                def body(i, carry):
                    off = tile_base + i * CH
                    goff = hd * e_pad + off if n_heads == 4 else off
                    pltpu.sync_copy(gidx_hbm.at[pl.ds(goff, CH)], gidx_v)
                    pltpu.sync_copy(alpha_hbm.at[pl.ds(goff, CH)], alpha_v)
                    pltpu.sync_copy(dst_hbm.at[pl.ds(off, CH)], dst_v)
                    pltpu.async_copy(table_hbm.at[gidx_v], rows_v, sem).wait()

                    def scale(g, carry2):
                        av = alpha_v[pl.ds(g * 16, 16)]
                        for j16 in range(16):
                            a = jnp.full((16,), av[j16], jnp.float32)
                            row = g * 16 + j16
                            for l in range(HID // 16):
                                sl = pl.ds(l * 16, 16)
                                rows_v[row, sl] = rows_v[row, sl] * a
                        return carry2

                    lax.fori_loop(0, CH // 16, scale, 0)
                    pltpu.sync_copy(rows_v, acc_sh.at[dst_v], add=True)
                    return carry

                lax.fori_loop(0, chunks, body, 0)
                plsc.subcore_barrier()
                if n_heads == 4:
                    out_base = hd * N_PAD + s * rows_pt
                else:
                    out_base = c * N_PAD + s * rows_pt
                pltpu.sync_copy(acc_sh.at[pl.ds(s * rows_pt, rows_pt)],
                                out_hbm.at[pl.ds(out_base, rows_pt)])
                plsc.subcore_barrier()

    return k(table, gidx_flat, alpha_flat, dst, zeros)


def _mm1(x, W1):
    nb = 10
    bn = N // nb

    def body(x_ref, w_ref, o_ref):
        o_ref[...] = jnp.dot(x_ref[...], w_ref[...],
                             preferred_element_type=jnp.float32)

    return pl.pallas_call(
        body,
        grid=(nb, 4),
        in_specs=[
            pl.BlockSpec((bn, F_IN), lambda i, h: (i, 0)),
            pl.BlockSpec((F_IN, HID), lambda i, h: (0, h)),
        ],
        out_specs=pl.BlockSpec((bn, HID), lambda i, h: (h * nb + i, 0)),
        out_shape=jax.ShapeDtypeStruct((4 * N, HID), jnp.float32),
    )(x, W1)


def _col_stats(y):
    nb = 10
    bn = N // nb
    C = y.shape[1]

    def body(y_ref, o_ref):
        @pl.when(pl.program_id(0) == 0)
        def _():
            o_ref[...] = jnp.zeros((8, C), jnp.float32)
        blk = y_ref[...]
        o_ref[0:1, :] = o_ref[0:1, :] + blk.sum(axis=0, keepdims=True)
        o_ref[1:2, :] = o_ref[1:2, :] + (blk * blk).sum(axis=0, keepdims=True)

    return pl.pallas_call(
        body,
        grid=(nb,),
        in_specs=[pl.BlockSpec((bn, C), lambda i: (i, 0))],
        out_specs=pl.BlockSpec((8, C), lambda i: (0, 0)),
        out_shape=jax.ShapeDtypeStruct((8, C), jnp.float32),
    )(y)


def _bn_relu_mm(y, scale, shift, W):
    nb = 10
    bn = N // nb
    C = y.shape[1]
    Cout = W.shape[1]

    def body(y_ref, s_ref, b_ref, w_ref, o_ref):
        h = jax.nn.relu(y_ref[...] * s_ref[...] + b_ref[...])
        o_ref[...] = jnp.dot(h, w_ref[...], preferred_element_type=jnp.float32)

    return pl.pallas_call(
        body,
        grid=(nb,),
        in_specs=[
            pl.BlockSpec((bn, C), lambda i: (i, 0)),
            pl.BlockSpec((1, C), lambda i: (0, 0)),
            pl.BlockSpec((1, C), lambda i: (0, 0)),
            pl.BlockSpec((C, Cout), lambda i: (0, 0)),
        ],
        out_specs=pl.BlockSpec((bn, Cout), lambda i: (i, 0)),
        out_shape=jax.ShapeDtypeStruct((N, Cout), jnp.float32),
    )(y, scale, shift, W)


def _bn_relu_gate_vals(y2, scale, shift, Wg_row, bg):
    nb = 10
    bn = N // nb

    def body(y_ref, s_ref, b_ref, wg_ref, bg_ref, oh_ref, os_ref):
        hc = jax.nn.relu(y_ref[...] * s_ref[...] + b_ref[...])
        gl = (hc * wg_ref[...]).sum(axis=1, keepdims=True) + bg_ref[...]
        eg = jnp.exp(gl)
        oh_ref[...] = eg * hc
        os_ref[...] = jnp.concatenate(
            [eg, jnp.zeros((bn, 127), jnp.float32)], axis=1)

    return pl.pallas_call(
        body,
        grid=(nb,),
        in_specs=[
            pl.BlockSpec((bn, HID), lambda i: (i, 0)),
            pl.BlockSpec((1, HID), lambda i: (0, 0)),
            pl.BlockSpec((1, HID), lambda i: (0, 0)),
            pl.BlockSpec((1, HID), lambda i: (0, 0)),
            pl.BlockSpec((1, 1), lambda i: (0, 0)),
        ],
        out_specs=(pl.BlockSpec((bn, HID), lambda i: (i, 0)),
                   pl.BlockSpec((bn, HID), lambda i: (i, 0))),
        out_shape=(jax.ShapeDtypeStruct((N, HID), jnp.float32),
                   jax.ShapeDtypeStruct((N, HID), jnp.float32)),
    )(y2, scale, shift, Wg_row, bg)


def _lstm_heads_body(xseq, gfeat, Wi1, Wh1, b1, Wi2, Wh2, b2,
                     Wo1, bo1, Wo2, bo2, Wb1, bb1, Wb2, bb2,
                     orange_ref, blue_ref):
    def step(xt, carry):
        h1, c1, h2, c2 = carry
        g1 = xt @ Wi1[...] + h1 @ Wh1[...] + b1[...]
        i1 = jax.nn.sigmoid(g1[:, 0 * LSTM_H:1 * LSTM_H])
        f1 = jax.nn.sigmoid(g1[:, 1 * LSTM_H:2 * LSTM_H])
        gg1 = jnp.tanh(g1[:, 2 * LSTM_H:3 * LSTM_H])
        o1 = jax.nn.sigmoid(g1[:, 3 * LSTM_H:4 * LSTM_H])
        c1 = f1 * c1 + i1 * gg1
        h1 = o1 * jnp.tanh(c1)
        g2 = h1 @ Wi2[...] + h2 @ Wh2[...] + b2[...]
        i2 = jax.nn.sigmoid(g2[:, 0 * LSTM_H:1 * LSTM_H])
        f2 = jax.nn.sigmoid(g2[:, 1 * LSTM_H:2 * LSTM_H])
        gg2 = jnp.tanh(g2[:, 2 * LSTM_H:3 * LSTM_H])
        o2 = jax.nn.sigmoid(g2[:, 3 * LSTM_H:4 * LSTM_H])
        c2 = f2 * c2 + i2 * gg2
        h2 = o2 * jnp.tanh(c2)
        return h1, c1, h2, c2

    z = jnp.zeros((BS, LSTM_H), dtype=jnp.float32)
    carry = (z, z, z, z)
    for t in range(SEQ):
        carry = step(xseq[t], carry)
    h1, c1, h2, c2 = carry
    comb = jnp.concatenate([h2, gfeat[...]], axis=1)
    ho = jax.nn.relu(comb @ Wo1[...] + bo1[...])
    hb = jax.nn.relu(comb @ Wb1[...] + bb1[...])
    orange_ref[...] = (ho * Wo2[...].reshape(1, -1)).sum(
        axis=1, keepdims=True) + bo2[...]
    blue_ref[...] = (hb * Wb2[...].reshape(1, -1)).sum(
        axis=1, keepdims=True) + bb2[...]


def _lstm_heads(seq, gfeat, Wi1, Wh1, b1, Wi2, Wh2, b2,
                Wo1, bo1, Wo2, bo2, Wb1, bb1, Wb2, bb2):
    out_shape = (jax.ShapeDtypeStruct((BS, 1), jnp.float32),
                 jax.ShapeDtypeStruct((BS, 1), jnp.float32))
    return pl.pallas_call(
        _lstm_heads_body,
        out_shape=out_shape,
    )(seq, gfeat, Wi1, Wh1, b1, Wi2, Wh2, b2,
      Wo1, bo1, Wo2, bo2, Wb1, bb1, Wb2, bb2)


def _score_fold(We, ae, heads, fout):
    return jnp.einsum('khf,hf->kh', We.reshape(We.shape[0], heads, fout), ae)


def _pad_rows(a, m):
    return jnp.pad(a, ((0, m - a.shape[0]),) + ((0, 0),) * (a.ndim - 1))


def kernel(x, edge_index, edge_attr, batch, global_features, W1, as1, ad1, We1, ae1, b1, g1, be1, W2, as2, ad2, We2, ae2, b2, g2, be2, Wg, bg, Wi1, Wh1, bi1, bh1, Wi2, Wh2, bi2, bh2, Wo1, bo1, Wo2, bo2, Wb1, bb1, Wb2, bb2):
    src, dst = edge_index[0], edge_index[1]
    dst_p32 = jnp.concatenate(
        [dst, jnp.arange(E_PAD32 - E, dtype=jnp.int32) % N])

    WeS1 = _score_fold(We1, ae1, 4, HID)
    WeS2 = _score_fold(We2, ae2, 1, HID)
    es = edge_attr @ jnp.concatenate([WeS1, WeS2], axis=1)
    es1, es2 = es[:, :4], es[:, 4:5]

    h1t = _mm1(x, W1)
    hh1 = h1t.reshape(4, N, HID)
    asrc1 = x @ jnp.einsum('khf,hf->kh', W1.reshape(F_IN, 4, HID), as1)
    adst1 = x @ jnp.einsum('khf,hf->kh', W1.reshape(F_IN, 4, HID), ad1)

    le1 = jax.nn.leaky_relu(asrc1[src] + adst1[dst] + es1, 0.2)
    ee1 = jnp.exp(le1)

    rows1 = jnp.concatenate(
        [jnp.ones((E, 1), jnp.float32), es1, es2, ee1,
         jnp.zeros((E, 118), jnp.float32)], axis=1)
    rows1 = _pad_rows(rows1, E_PAD32)
    zeros128 = jnp.zeros((N_PAD // NS, HID), jnp.float32)
    st = _sc_segsum(rows1, dst_p32, zeros128, n_pad=N_PAD)
    st = (st[0] + st[1])[:N]
    deg, esum1, esum2, densum1 = st[:, 0], st[:, 1:5], st[:, 5:6], st[:, 6:10]
    invdeg = 1.0 / jnp.maximum(deg, 1.0)
    ls1 = esum1 * invdeg[:, None]
    ls2 = esum2 * invdeg[:, None]

    ef1 = jnp.exp(jax.nn.leaky_relu(asrc1 + adst1 + ls1, 0.2))
    denom1 = densum1 + ef1 + 1e-16
    alpha1 = ee1 / denom1[dst]
    alpha_f1 = ef1 / denom1

    pad16 = jnp.arange(E_PAD16 - E, dtype=jnp.int32) % N
    src_p16 = jnp.concatenate([src, pad16])
    gidx1 = (src_p16[None, :]
             + (jnp.arange(4, dtype=jnp.int32) * N)[:, None]).reshape(-1)
    alpha1f = jnp.pad(alpha1.T, ((0, 0), (0, E_PAD16 - E))).reshape(-1)
    dst_p16 = jnp.concatenate([dst, pad16])
    agg1 = _sc_aggregate(h1t, gidx1, alpha1f, dst_p16, zeros128, n_heads=4)
    agg1 = agg1.reshape(4, N_PAD, HID)[:, :N]
    out1 = agg1 + hh1 * alpha_f1.T[:, :, None]
    y1 = out1.transpose(1, 0, 2).reshape(N, 4 * HID) + b1

    s1 = _col_stats(y1)
    mu1 = s1[0] / N
    var1 = s1[1] / N - mu1 * mu1
    sc1 = g1 / jnp.sqrt(var1 + 1e-5)
    sh1 = be1 - mu1 * sc1
    h2 = _bn_relu_mm(y1, sc1.reshape(1, -1), sh1.reshape(1, -1), W2)

    asrc2 = (h2 * as2).sum(1, keepdims=True)
    adst2 = (h2 * ad2).sum(1, keepdims=True)
    le2 = jax.nn.leaky_relu(asrc2[src] + adst2[dst] + es2, 0.2)
    ee2 = jnp.exp(le2)

    rows2 = _pad_rows(jnp.concatenate(
        [ee2, jnp.zeros((E, 127), jnp.float32)], axis=1), E_PAD32)
    d2 = _sc_segsum(rows2, dst_p32, zeros128, n_pad=N_PAD)
    densum2 = (d2[0] + d2[1])[:N, 0:1]
    ef2 = jnp.exp(jax.nn.leaky_relu(asrc2 + adst2 + ls2, 0.2))
    denom2 = densum2 + ef2 + 1e-16
    alpha2 = ee2 / denom2[dst]
    alpha_f2 = ef2 / denom2

    gidx2 = jnp.concatenate(
        [src, jnp.arange(E_PAD32 - E, dtype=jnp.int32) % N])
    alpha2f = jnp.pad(alpha2[:, 0], (0, E_PAD32 - E))
    agg2 = _sc_aggregate(h2, gidx2, alpha2f, dst_p32, zeros128, n_heads=1)
    agg2 = agg2.reshape(2, N_PAD, HID)
    y2 = agg2[0, :N] + agg2[1, :N] + h2 * alpha_f2 + b2

    s2 = _col_stats(y2)
    mu2 = s2[0] / N
    var2 = s2[1] / N - mu2 * mu2
    sc2 = g2 / jnp.sqrt(var2 + 1e-5)
    sh2 = be2 - mu2 * sc2
    pvh, pvs = _bn_relu_gate_vals(y2, sc2.reshape(1, -1), sh2.reshape(1, -1),
                                  Wg.reshape(1, -1), bg.reshape(1, 1))
    pvh = _pad_rows(pvh, NP_PAD32)
    pvs = _pad_rows(pvs, NP_PAD32)
    batch_p = jnp.concatenate(
        [batch, jnp.arange(NP_PAD32 - N, dtype=jnp.int32) % NG])
    zpg = jnp.zeros((NG_PAD // NS, HID), jnp.float32)
    ph = _sc_segsum(pvh, batch_p, zpg, n_pad=NG_PAD)
    ps = _sc_segsum(pvs, batch_p, zpg, n_pad=NG_PAD)
    ph = (ph[0] + ph[1])[:NG]
    ps = (ps[0] + ps[1])[:NG, 0:1]
    graph_embeds = ph / (ps + 1e-16)

    seq = graph_embeds.reshape(BS, SEQ, HID).transpose(1, 0, 2)
    gfeat = global_features[SEQ - 1::SEQ]
    orange, blue = _lstm_heads(
        seq, gfeat, Wi1, Wh1, (bi1 + bh1).reshape(1, -1),
        Wi2, Wh2, (bi2 + bh2).reshape(1, -1),
        Wo1, bo1.reshape(1, -1), Wo2, bo2.reshape(1, 1),
        Wb1, bb1.reshape(1, -1), Wb2, bb2.reshape(1, 1))
    return (orange, blue)


# speedup vs baseline: 16.3255x; 2.8335x over previous
import functools
import jax
import jax.numpy as jnp
from jax import lax
from jax.experimental import pallas as pl
from jax.experimental.pallas import tpu as pltpu
from jax.experimental.pallas import tpu_sc as plsc

N = 10000
E = 160000
F_IN = 256
EDGE_DIM = 16
HID = 128
LSTM_H = 256
GFEAT = 32
NG = 240
SEQ = 6
BS = NG // SEQ

NC = 2
NS = 16
CH = 256

N_PAD = 10112
E_PAD16 = 163840
E_PAD32 = 163840
NP_PAD32 = 16384
NG_PAD = 256
DUPN = 320000


def _ceil_to(x, m):
    return (x + m - 1) // m * m


@functools.partial(jax.jit, static_argnames=("n_pad",))
def _sc_segsum(vals, idx, zeros, *, n_pad):
    M, W = vals.shape
    per_tile = M // (NC * NS)
    chunks = per_tile // CH
    rows_pt = n_pad // NS
    mesh = plsc.VectorSubcoreMesh(core_axis_name="c", subcore_axis_name="s")

    @functools.partial(
        pl.kernel, mesh=mesh,
        out_type=jax.ShapeDtypeStruct((NC * n_pad, W), jnp.float32),
        scratch_types=[
            pltpu.VMEM((CH,), jnp.int32),
            pltpu.VMEM((CH, W), jnp.float32),
            pltpu.VMEM_SHARED((n_pad, W), jnp.float32),
        ],
    )
    def k(vals_hbm, idx_hbm, zeros_hbm, out_hbm, idx_v, val_v, acc_sh):
        c = lax.axis_index("c")
        s = lax.axis_index("s")
        wid = s * NC + c

        if True:
            pltpu.sync_copy(zeros_hbm, acc_sh.at[pl.ds(s * rows_pt, rows_pt)])
            plsc.subcore_barrier()

            def body(i, carry):
                off = wid * per_tile + i * CH
                pltpu.sync_copy(idx_hbm.at[pl.ds(off, CH)], idx_v)
                pltpu.sync_copy(vals_hbm.at[pl.ds(off, CH)], val_v)
                pltpu.sync_copy(val_v, acc_sh.at[idx_v], add=True)
                return carry

            lax.fori_loop(0, chunks, body, 0)
            plsc.subcore_barrier()
            pltpu.sync_copy(acc_sh.at[pl.ds(s * rows_pt, rows_pt)],
                            out_hbm.at[pl.ds(c * n_pad + s * rows_pt, rows_pt)])

    return k(vals, idx, zeros).reshape(NC, n_pad, W)


def _sc_aggregate(table, gidx_flat, alpha_flat, dst, zeros, *, n_heads):
    if n_heads == 4:
        e_pad, heads_per_core, n_out_maj = E_PAD16, 2, 4
        per_tile = e_pad // NS
    else:
        e_pad, heads_per_core, n_out_maj = E_PAD32, 1, 2
        per_tile = e_pad // (NC * NS)
    chunks = per_tile // CH
    rows_pt = N_PAD // NS
    mesh = plsc.VectorSubcoreMesh(core_axis_name="c", subcore_axis_name="s")

    @functools.partial(
        pl.kernel, mesh=mesh,
        out_type=jax.ShapeDtypeStruct((n_out_maj * N_PAD, HID), jnp.float32),
        scratch_types=[
            pltpu.VMEM((CH,), jnp.int32),
            pltpu.VMEM((CH,), jnp.int32),
            pltpu.VMEM((CH,), jnp.float32),
            pltpu.VMEM((CH, HID), jnp.float32),
            pltpu.VMEM_SHARED((N_PAD, HID), jnp.float32),
            pltpu.SemaphoreType.DMA,
        ],
    )
    def k(table_hbm, gidx_hbm, alpha_hbm, dst_hbm, zeros_hbm, out_hbm,
          gidx_v, dst_v, alpha_v, rows_v, acc_sh, sem):
        c = lax.axis_index("c")
        s = lax.axis_index("s")

        if True:
            for klocal in range(heads_per_core):
                hd = c * heads_per_core + klocal
                pltpu.sync_copy(zeros_hbm,
                                acc_sh.at[pl.ds(s * rows_pt, rows_pt)])
                plsc.subcore_barrier()

                if n_heads == 4:
                    tile_base = s * per_tile
                else:
                    tile_base = (s * NC + c) * per_tile

                def body(i, carry):
                    off = tile_base + i * CH
                    goff = hd * e_pad + off if n_heads == 4 else off
                    pltpu.sync_copy(gidx_hbm.at[pl.ds(goff, CH)], gidx_v)
                    pltpu.sync_copy(alpha_hbm.at[pl.ds(goff, CH)], alpha_v)
                    pltpu.sync_copy(dst_hbm.at[pl.ds(off, CH)], dst_v)
                    pltpu.async_copy(table_hbm.at[gidx_v], rows_v, sem).wait()

                    def scale(g, carry2):
                        av = alpha_v[pl.ds(g * 16, 16)]
                        for j16 in range(16):
                            a = jnp.full((16,), av[j16], jnp.float32)
                            row = g * 16 + j16
                            for l in range(HID // 16):
                                sl = pl.ds(l * 16, 16)
                                rows_v[row, sl] = rows_v[row, sl] * a
                        return carry2

                    lax.fori_loop(0, CH // 16, scale, 0)
                    pltpu.sync_copy(rows_v, acc_sh.at[dst_v], add=True)
                    return carry

                lax.fori_loop(0, chunks, body, 0)
                plsc.subcore_barrier()
                if n_heads == 4:
                    out_base = hd * N_PAD + s * rows_pt
                else:
                    out_base = c * N_PAD + s * rows_pt
                pltpu.sync_copy(acc_sh.at[pl.ds(s * rows_pt, rows_pt)],
                                out_hbm.at[pl.ds(out_base, rows_pt)])
                plsc.subcore_barrier()

    return k(table, gidx_flat, alpha_flat, dst, zeros)


def _mm1(x, W1):
    nb = 10
    bn = N // nb

    def body(x_ref, w_ref, o_ref):
        o_ref[...] = jnp.dot(x_ref[...], w_ref[...],
                             preferred_element_type=jnp.float32)

    return pl.pallas_call(
        body,
        grid=(nb, 4),
        in_specs=[
            pl.BlockSpec((bn, F_IN), lambda i, h: (i, 0)),
            pl.BlockSpec((F_IN, HID), lambda i, h: (0, h)),
        ],
        out_specs=pl.BlockSpec((bn, HID), lambda i, h: (h * nb + i, 0)),
        out_shape=jax.ShapeDtypeStruct((4 * N, HID), jnp.float32),
    )(x, W1)


def _col_stats(y):
    nb = 10
    bn = N // nb
    C = y.shape[1]

    def body(y_ref, o_ref):
        @pl.when(pl.program_id(0) == 0)
        def _():
            o_ref[...] = jnp.zeros((8, C), jnp.float32)
        blk = y_ref[...]
        o_ref[0:1, :] = o_ref[0:1, :] + blk.sum(axis=0, keepdims=True)
        o_ref[1:2, :] = o_ref[1:2, :] + (blk * blk).sum(axis=0, keepdims=True)

    return pl.pallas_call(
        body,
        grid=(nb,),
        in_specs=[pl.BlockSpec((bn, C), lambda i: (i, 0))],
        out_specs=pl.BlockSpec((8, C), lambda i: (0, 0)),
        out_shape=jax.ShapeDtypeStruct((8, C), jnp.float32),
    )(y)


def _bn_relu_mm(y, scale, shift, W):
    nb = 10
    bn = N // nb
    C = y.shape[1]
    Cout = W.shape[1]

    def body(y_ref, s_ref, b_ref, w_ref, o_ref):
        h = jax.nn.relu(y_ref[...] * s_ref[...] + b_ref[...])
        o_ref[...] = jnp.dot(h, w_ref[...], preferred_element_type=jnp.float32)

    return pl.pallas_call(
        body,
        grid=(nb,),
        in_specs=[
            pl.BlockSpec((bn, C), lambda i: (i, 0)),
            pl.BlockSpec((1, C), lambda i: (0, 0)),
            pl.BlockSpec((1, C), lambda i: (0, 0)),
            pl.BlockSpec((C, Cout), lambda i: (0, 0)),
        ],
        out_specs=pl.BlockSpec((bn, Cout), lambda i: (i, 0)),
        out_shape=jax.ShapeDtypeStruct((N, Cout), jnp.float32),
    )(y, scale, shift, W)


def _bn_relu_gate_vals(y2, scale, shift, Wg_row, bg):
    nb = 10
    bn = N // nb

    def body(y_ref, s_ref, b_ref, wg_ref, bg_ref, oh_ref, os_ref):
        hc = jax.nn.relu(y_ref[...] * s_ref[...] + b_ref[...])
        gl = (hc * wg_ref[...]).sum(axis=1, keepdims=True) + bg_ref[...]
        eg = jnp.exp(gl)
        oh_ref[...] = eg * hc
        os_ref[...] = jnp.concatenate(
            [eg, jnp.zeros((bn, 127), jnp.float32)], axis=1)

    return pl.pallas_call(
        body,
        grid=(nb,),
        in_specs=[
            pl.BlockSpec((bn, HID), lambda i: (i, 0)),
            pl.BlockSpec((1, HID), lambda i: (0, 0)),
            pl.BlockSpec((1, HID), lambda i: (0, 0)),
            pl.BlockSpec((1, HID), lambda i: (0, 0)),
            pl.BlockSpec((1, 1), lambda i: (0, 0)),
        ],
        out_specs=(pl.BlockSpec((bn, HID), lambda i: (i, 0)),
                   pl.BlockSpec((bn, HID), lambda i: (i, 0))),
        out_shape=(jax.ShapeDtypeStruct((N, HID), jnp.float32),
                   jax.ShapeDtypeStruct((N, HID), jnp.float32)),
    )(y2, scale, shift, Wg_row, bg)


def _lstm_heads_body(xseq, gfeat, Wi1, Wh1, b1, Wi2, Wh2, b2,
                     Wo1, bo1, Wo2, bo2, Wb1, bb1, Wb2, bb2,
                     orange_ref, blue_ref):
    def step(xt, carry):
        h1, c1, h2, c2 = carry
        g1 = xt @ Wi1[...] + h1 @ Wh1[...] + b1[...]
        i1 = jax.nn.sigmoid(g1[:, 0 * LSTM_H:1 * LSTM_H])
        f1 = jax.nn.sigmoid(g1[:, 1 * LSTM_H:2 * LSTM_H])
        gg1 = jnp.tanh(g1[:, 2 * LSTM_H:3 * LSTM_H])
        o1 = jax.nn.sigmoid(g1[:, 3 * LSTM_H:4 * LSTM_H])
        c1 = f1 * c1 + i1 * gg1
        h1 = o1 * jnp.tanh(c1)
        g2 = h1 @ Wi2[...] + h2 @ Wh2[...] + b2[...]
        i2 = jax.nn.sigmoid(g2[:, 0 * LSTM_H:1 * LSTM_H])
        f2 = jax.nn.sigmoid(g2[:, 1 * LSTM_H:2 * LSTM_H])
        gg2 = jnp.tanh(g2[:, 2 * LSTM_H:3 * LSTM_H])
        o2 = jax.nn.sigmoid(g2[:, 3 * LSTM_H:4 * LSTM_H])
        c2 = f2 * c2 + i2 * gg2
        h2 = o2 * jnp.tanh(c2)
        return h1, c1, h2, c2

    z = jnp.zeros((BS, LSTM_H), dtype=jnp.float32)
    carry = (z, z, z, z)
    for t in range(SEQ):
        carry = step(xseq[t], carry)
    h1, c1, h2, c2 = carry
    comb = jnp.concatenate([h2, gfeat[...]], axis=1)
    ho = jax.nn.relu(comb @ Wo1[...] + bo1[...])
    hb = jax.nn.relu(comb @ Wb1[...] + bb1[...])
    orange_ref[...] = (ho * Wo2[...].reshape(1, -1)).sum(
        axis=1, keepdims=True) + bo2[...]
    blue_ref[...] = (hb * Wb2[...].reshape(1, -1)).sum(
        axis=1, keepdims=True) + bb2[...]


def _lstm_heads(seq, gfeat, Wi1, Wh1, b1, Wi2, Wh2, b2,
                Wo1, bo1, Wo2, bo2, Wb1, bb1, Wb2, bb2):
    out_shape = (jax.ShapeDtypeStruct((BS, 1), jnp.float32),
                 jax.ShapeDtypeStruct((BS, 1), jnp.float32))
    return pl.pallas_call(
        _lstm_heads_body,
        out_shape=out_shape,
    )(seq, gfeat, Wi1, Wh1, b1, Wi2, Wh2, b2,
      Wo1, bo1, Wo2, bo2, Wb1, bb1, Wb2, bb2)


def _score_fold(We, ae, heads, fout):
    return jnp.einsum('khf,hf->kh', We.reshape(We.shape[0], heads, fout), ae)


def _pad_rows(a, m):
    return jnp.pad(a, ((0, m - a.shape[0]),) + ((0, 0),) * (a.ndim - 1))


def kernel(x, edge_index, edge_attr, batch, global_features, W1, as1, ad1, We1, ae1, b1, g1, be1, W2, as2, ad2, We2, ae2, b2, g2, be2, Wg, bg, Wi1, Wh1, bi1, bh1, Wi2, Wh2, bi2, bh2, Wo1, bo1, Wo2, bo2, Wb1, bb1, Wb2, bb2):
    src, dst = edge_index[0], edge_index[1]
    dst_p32 = jnp.concatenate(
        [dst, jnp.arange(E_PAD32 - E, dtype=jnp.int32) % N])

    WeS1 = _score_fold(We1, ae1, 4, HID)
    WeS2 = _score_fold(We2, ae2, 1, HID)
    es = edge_attr @ jnp.concatenate([WeS1, WeS2], axis=1)
    es1, es2 = es[:, :4], es[:, 4:5]

    h1t = _mm1(x, W1)
    hh1 = h1t.reshape(4, N, HID)
    asrc1 = x @ jnp.einsum('khf,hf->kh', W1.reshape(F_IN, 4, HID), as1)
    adst1 = x @ jnp.einsum('khf,hf->kh', W1.reshape(F_IN, 4, HID), ad1)

    scores1 = jnp.pad(jnp.concatenate([asrc1, adst1], axis=1),
                      ((0, DUPN - N), (0, 0)))
    ga1 = scores1[src][:, 0:4]
    gd1 = scores1[dst][:, 4:8]
    ee1 = jnp.exp(jax.nn.leaky_relu(ga1 + gd1 + es1, 0.2))
    pad16 = jnp.arange(E_PAD16 - E, dtype=jnp.int32) % N
    src_p16 = jnp.concatenate([src, pad16])
    dst_p16 = jnp.concatenate([dst, pad16])
    hoff = (jnp.arange(4, dtype=jnp.int32) * N)[:, None]
    gidx1 = (src_p16[None, :] + hoff).reshape(-1)

    rows1 = jnp.concatenate(
        [jnp.ones((E, 1), jnp.float32), es1, es2, ee1,
         jnp.zeros((E, 118), jnp.float32)], axis=1)
    rows1 = _pad_rows(rows1, E_PAD32)
    zeros128 = jnp.zeros((N_PAD // NS, HID), jnp.float32)
    st = _sc_segsum(rows1, dst_p32, zeros128, n_pad=N_PAD)
    st = (st[0] + st[1])[:N]
    deg, esum1, esum2, densum1 = st[:, 0], st[:, 1:5], st[:, 5:6], st[:, 6:10]
    invdeg = 1.0 / jnp.maximum(deg, 1.0)
    ls1 = esum1 * invdeg[:, None]
    ls2 = esum2 * invdeg[:, None]

    ef1 = jnp.exp(jax.nn.leaky_relu(asrc1 + adst1 + ls1, 0.2))
    denom1 = densum1 + ef1 + 1e-16
    alpha_f1 = ef1 / denom1
    rden1 = 1.0 / denom1

    ee1f = jnp.pad(ee1.T, ((0, 0), (0, E_PAD16 - E))).reshape(-1)
    agg1 = _sc_aggregate(h1t, gidx1, ee1f, dst_p16, zeros128, n_heads=4)
    agg1 = agg1.reshape(4, N_PAD, HID)[:, :N]
    out1 = agg1 * rden1.T[:, :, None] + hh1 * alpha_f1.T[:, :, None]
    y1 = out1.transpose(1, 0, 2).reshape(N, 4 * HID) + b1

    s1 = _col_stats(y1)
    mu1 = s1[0] / N
    var1 = s1[1] / N - mu1 * mu1
    sc1 = g1 / jnp.sqrt(var1 + 1e-5)
    sh1 = be1 - mu1 * sc1
    h2 = _bn_relu_mm(y1, sc1.reshape(1, -1), sh1.reshape(1, -1), W2)

    asrc2 = (h2 * as2).sum(1, keepdims=True)
    adst2 = (h2 * ad2).sum(1, keepdims=True)
    gidx2 = jnp.concatenate(
        [src, jnp.arange(E_PAD32 - E, dtype=jnp.int32) % N])
    scores2 = jnp.pad(jnp.concatenate(
        [asrc2, adst2, jnp.zeros((N, 6), jnp.float32)], axis=1),
        ((0, DUPN - N), (0, 0)))
    ga2 = scores2[src][:, 0:1]
    gd2 = scores2[dst][:, 1:2]
    ee2 = jnp.exp(jax.nn.leaky_relu(ga2 + gd2 + es2, 0.2))
    ee2f = jnp.pad(ee2[:, 0], (0, E_PAD32 - E))

    rows2 = jnp.concatenate(
        [ee2f[:, None], jnp.zeros((E_PAD32, 127), jnp.float32)], axis=1)
    d2 = _sc_segsum(rows2, dst_p32, zeros128, n_pad=N_PAD)
    densum2 = (d2[0] + d2[1])[:N, 0:1]
    ef2 = jnp.exp(jax.nn.leaky_relu(asrc2 + adst2 + ls2, 0.2))
    denom2 = densum2 + ef2 + 1e-16
    alpha_f2 = ef2 / denom2

    agg2 = _sc_aggregate(h2, gidx2, ee2f, dst_p32, zeros128, n_heads=1)
    agg2 = agg2.reshape(2, N_PAD, HID)
    y2 = (agg2[0, :N] + agg2[1, :N]) / denom2 + h2 * alpha_f2 + b2

    s2 = _col_stats(y2)
    mu2 = s2[0] / N
    var2 = s2[1] / N - mu2 * mu2
    sc2 = g2 / jnp.sqrt(var2 + 1e-5)
    sh2 = be2 - mu2 * sc2
    pvh, pvs = _bn_relu_gate_vals(y2, sc2.reshape(1, -1), sh2.reshape(1, -1),
                                  Wg.reshape(1, -1), bg.reshape(1, 1))
    pvh = _pad_rows(pvh, NP_PAD32)
    pvs = _pad_rows(pvs, NP_PAD32)
    batch_p = jnp.concatenate(
        [batch, jnp.arange(NP_PAD32 - N, dtype=jnp.int32) % NG])
    zpg = jnp.zeros((NG_PAD // NS, HID), jnp.float32)
    ph = _sc_segsum(pvh, batch_p, zpg, n_pad=NG_PAD)
    ps = _sc_segsum(pvs, batch_p, zpg, n_pad=NG_PAD)
    ph = (ph[0] + ph[1])[:NG]
    ps = (ps[0] + ps[1])[:NG, 0:1]
    graph_embeds = ph / (ps + 1e-16)

    seq = graph_embeds.reshape(BS, SEQ, HID).transpose(1, 0, 2)
    gfeat = global_features[SEQ - 1::SEQ]
    orange, blue = _lstm_heads(
        seq, gfeat, Wi1, Wh1, (bi1 + bh1).reshape(1, -1),
        Wi2, Wh2, (bi2 + bh2).reshape(1, -1),
        Wo1, bo1.reshape(1, -1), Wo2, bo2.reshape(1, 1),
        Wb1, bb1.reshape(1, -1), Wb2, bb2.reshape(1, 1))
    return (orange, blue)


# speedup vs baseline: 17.3121x; 1.0604x over previous
import functools
import jax
import jax.numpy as jnp
from jax import lax
from jax.experimental import pallas as pl
from jax.experimental.pallas import tpu as pltpu
from jax.experimental.pallas import tpu_sc as plsc

N = 10000
E = 160000
F_IN = 256
EDGE_DIM = 16
HID = 128
LSTM_H = 256
GFEAT = 32
NG = 240
SEQ = 6
BS = NG // SEQ

NC = 2
NS = 16
CH = 256

N_PAD = 10112
E_PAD16 = 163840
E_PAD32 = 163840
NP_PAD32 = 16384
NG_PAD = 256
DUPN = 320000


def _ceil_to(x, m):
    return (x + m - 1) // m * m


@functools.partial(jax.jit, static_argnames=("n_pad",))
def _sc_segsum(vals, idx, zeros, *, n_pad):
    M, W = vals.shape
    per_tile = M // (NC * NS)
    chunks = per_tile // CH
    rows_pt = n_pad // NS
    mesh = plsc.VectorSubcoreMesh(core_axis_name="c", subcore_axis_name="s")

    @functools.partial(
        pl.kernel, mesh=mesh,
        out_type=jax.ShapeDtypeStruct((NC * n_pad, W), jnp.float32),
        scratch_types=[
            pltpu.VMEM((CH,), jnp.int32),
            pltpu.VMEM((CH, W), jnp.float32),
            pltpu.VMEM_SHARED((n_pad, W), jnp.float32),
            pltpu.SemaphoreType.DMA,
        ],
    )
    def k(vals_hbm, idx_hbm, zeros_hbm, out_hbm, idx_v, val_v, acc_sh, sem):
        c = lax.axis_index("c")
        s = lax.axis_index("s")
        wid = s * NC + c

        if True:
            pltpu.sync_copy(zeros_hbm, acc_sh.at[pl.ds(s * rows_pt, rows_pt)])
            plsc.subcore_barrier()

            def body(i, carry):
                off = wid * per_tile + i * CH
                c1 = pltpu.async_copy(idx_hbm.at[pl.ds(off, CH)], idx_v, sem)
                c2 = pltpu.async_copy(vals_hbm.at[pl.ds(off, CH)], val_v, sem)
                c1.wait()
                c2.wait()
                pltpu.sync_copy(val_v, acc_sh.at[idx_v], add=True)
                return carry

            lax.fori_loop(0, chunks, body, 0)
            plsc.subcore_barrier()
            pltpu.sync_copy(acc_sh.at[pl.ds(s * rows_pt, rows_pt)],
                            out_hbm.at[pl.ds(c * n_pad + s * rows_pt, rows_pt)])

    return k(vals, idx, zeros).reshape(NC, n_pad, W)


def _sc_aggregate(table, gidx_flat, alpha_flat, dst, zeros, *, n_heads):
    if n_heads == 4:
        e_pad, heads_per_core, n_out_maj = E_PAD16, 2, 4
        per_tile = e_pad // NS
    else:
        e_pad, heads_per_core, n_out_maj = E_PAD32, 1, 2
        per_tile = e_pad // (NC * NS)
    chunks = per_tile // CH
    rows_pt = N_PAD // NS
    mesh = plsc.VectorSubcoreMesh(core_axis_name="c", subcore_axis_name="s")

    @functools.partial(
        pl.kernel, mesh=mesh,
        out_type=jax.ShapeDtypeStruct((n_out_maj * N_PAD, HID), jnp.float32),
        scratch_types=[
            pltpu.VMEM((CH,), jnp.int32),
            pltpu.VMEM((CH,), jnp.int32),
            pltpu.VMEM((CH,), jnp.float32),
            pltpu.VMEM((CH, HID), jnp.float32),
            pltpu.VMEM_SHARED((N_PAD, HID), jnp.float32),
            pltpu.SemaphoreType.DMA,
        ],
    )
    def k(table_hbm, gidx_hbm, alpha_hbm, dst_hbm, zeros_hbm, out_hbm,
          gidx_v, dst_v, alpha_v, rows_v, acc_sh, sem):
        c = lax.axis_index("c")
        s = lax.axis_index("s")

        if True:
            for klocal in range(heads_per_core):
                hd = c * heads_per_core + klocal
                pltpu.sync_copy(zeros_hbm,
                                acc_sh.at[pl.ds(s * rows_pt, rows_pt)])
                plsc.subcore_barrier()

                if n_heads == 4:
                    tile_base = s * per_tile
                else:
                    tile_base = (s * NC + c) * per_tile

                def body(i, carry):
                    off = tile_base + i * CH
                    goff = hd * e_pad + off if n_heads == 4 else off
                    c1 = pltpu.async_copy(gidx_hbm.at[pl.ds(goff, CH)],
                                          gidx_v, sem)
                    c2 = pltpu.async_copy(alpha_hbm.at[pl.ds(goff, CH)],
                                          alpha_v, sem)
                    c3 = pltpu.async_copy(dst_hbm.at[pl.ds(off, CH)],
                                          dst_v, sem)
                    c1.wait()
                    c2.wait()
                    c3.wait()
                    pltpu.async_copy(table_hbm.at[gidx_v], rows_v, sem).wait()

                    def scale(g, carry2):
                        av = alpha_v[pl.ds(g * 16, 16)]
                        for j16 in range(16):
                            a = jnp.full((16,), av[j16], jnp.float32)
                            row = g * 16 + j16
                            for l in range(HID // 16):
                                sl = pl.ds(l * 16, 16)
                                rows_v[row, sl] = rows_v[row, sl] * a
                        return carry2

                    lax.fori_loop(0, CH // 16, scale, 0)
                    pltpu.sync_copy(rows_v, acc_sh.at[dst_v], add=True)
                    return carry

                lax.fori_loop(0, chunks, body, 0)
                plsc.subcore_barrier()
                if n_heads == 4:
                    out_base = hd * N_PAD + s * rows_pt
                else:
                    out_base = c * N_PAD + s * rows_pt
                pltpu.sync_copy(acc_sh.at[pl.ds(s * rows_pt, rows_pt)],
                                out_hbm.at[pl.ds(out_base, rows_pt)])
                plsc.subcore_barrier()

    return k(table, gidx_flat, alpha_flat, dst, zeros)


def _mm1(x, W1):
    nb = 10
    bn = N // nb

    def body(x_ref, w_ref, o_ref):
        o_ref[...] = jnp.dot(x_ref[...], w_ref[...],
                             preferred_element_type=jnp.float32)

    return pl.pallas_call(
        body,
        grid=(nb, 4),
        in_specs=[
            pl.BlockSpec((bn, F_IN), lambda i, h: (i, 0)),
            pl.BlockSpec((F_IN, HID), lambda i, h: (0, h)),
        ],
        out_specs=pl.BlockSpec((bn, HID), lambda i, h: (h * nb + i, 0)),
        out_shape=jax.ShapeDtypeStruct((4 * N, HID), jnp.float32),
    )(x, W1)


def _col_stats(y):
    nb = 10
    bn = N // nb
    C = y.shape[1]

    def body(y_ref, o_ref):
        @pl.when(pl.program_id(0) == 0)
        def _():
            o_ref[...] = jnp.zeros((8, C), jnp.float32)
        blk = y_ref[...]
        o_ref[0:1, :] = o_ref[0:1, :] + blk.sum(axis=0, keepdims=True)
        o_ref[1:2, :] = o_ref[1:2, :] + (blk * blk).sum(axis=0, keepdims=True)

    return pl.pallas_call(
        body,
        grid=(nb,),
        in_specs=[pl.BlockSpec((bn, C), lambda i: (i, 0))],
        out_specs=pl.BlockSpec((8, C), lambda i: (0, 0)),
        out_shape=jax.ShapeDtypeStruct((8, C), jnp.float32),
    )(y)


def _bn_relu_mm(y, scale, shift, W):
    nb = 10
    bn = N // nb
    C = y.shape[1]
    Cout = W.shape[1]

    def body(y_ref, s_ref, b_ref, w_ref, o_ref):
        h = jax.nn.relu(y_ref[...] * s_ref[...] + b_ref[...])
        o_ref[...] = jnp.dot(h, w_ref[...], preferred_element_type=jnp.float32)

    return pl.pallas_call(
        body,
        grid=(nb,),
        in_specs=[
            pl.BlockSpec((bn, C), lambda i: (i, 0)),
            pl.BlockSpec((1, C), lambda i: (0, 0)),
            pl.BlockSpec((1, C), lambda i: (0, 0)),
            pl.BlockSpec((C, Cout), lambda i: (0, 0)),
        ],
        out_specs=pl.BlockSpec((bn, Cout), lambda i: (i, 0)),
        out_shape=jax.ShapeDtypeStruct((N, Cout), jnp.float32),
    )(y, scale, shift, W)


def _bn_relu_gate_vals(y2, scale, shift, Wg_row, bg):
    nb = 10
    bn = N // nb

    def body(y_ref, s_ref, b_ref, wg_ref, bg_ref, oh_ref, os_ref):
        hc = jax.nn.relu(y_ref[...] * s_ref[...] + b_ref[...])
        gl = (hc * wg_ref[...]).sum(axis=1, keepdims=True) + bg_ref[...]
        eg = jnp.exp(gl)
        oh_ref[...] = eg * hc
        os_ref[...] = jnp.concatenate(
            [eg, jnp.zeros((bn, 127), jnp.float32)], axis=1)

    return pl.pallas_call(
        body,
        grid=(nb,),
        in_specs=[
            pl.BlockSpec((bn, HID), lambda i: (i, 0)),
            pl.BlockSpec((1, HID), lambda i: (0, 0)),
            pl.BlockSpec((1, HID), lambda i: (0, 0)),
            pl.BlockSpec((1, HID), lambda i: (0, 0)),
            pl.BlockSpec((1, 1), lambda i: (0, 0)),
        ],
        out_specs=(pl.BlockSpec((bn, HID), lambda i: (i, 0)),
                   pl.BlockSpec((bn, HID), lambda i: (i, 0))),
        out_shape=(jax.ShapeDtypeStruct((N, HID), jnp.float32),
                   jax.ShapeDtypeStruct((N, HID), jnp.float32)),
    )(y2, scale, shift, Wg_row, bg)


def _lstm_heads_body(xseq, gfeat, Wi1, Wh1, b1, Wi2, Wh2, b2,
                     Wo1, bo1, Wo2, bo2, Wb1, bb1, Wb2, bb2,
                     orange_ref, blue_ref):
    def step(xt, carry):
        h1, c1, h2, c2 = carry
        g1 = xt @ Wi1[...] + h1 @ Wh1[...] + b1[...]
        i1 = jax.nn.sigmoid(g1[:, 0 * LSTM_H:1 * LSTM_H])
        f1 = jax.nn.sigmoid(g1[:, 1 * LSTM_H:2 * LSTM_H])
        gg1 = jnp.tanh(g1[:, 2 * LSTM_H:3 * LSTM_H])
        o1 = jax.nn.sigmoid(g1[:, 3 * LSTM_H:4 * LSTM_H])
        c1 = f1 * c1 + i1 * gg1
        h1 = o1 * jnp.tanh(c1)
        g2 = h1 @ Wi2[...] + h2 @ Wh2[...] + b2[...]
        i2 = jax.nn.sigmoid(g2[:, 0 * LSTM_H:1 * LSTM_H])
        f2 = jax.nn.sigmoid(g2[:, 1 * LSTM_H:2 * LSTM_H])
        gg2 = jnp.tanh(g2[:, 2 * LSTM_H:3 * LSTM_H])
        o2 = jax.nn.sigmoid(g2[:, 3 * LSTM_H:4 * LSTM_H])
        c2 = f2 * c2 + i2 * gg2
        h2 = o2 * jnp.tanh(c2)
        return h1, c1, h2, c2

    z = jnp.zeros((BS, LSTM_H), dtype=jnp.float32)
    carry = (z, z, z, z)
    for t in range(SEQ):
        carry = step(xseq[t], carry)
    h1, c1, h2, c2 = carry
    comb = jnp.concatenate([h2, gfeat[...]], axis=1)
    ho = jax.nn.relu(comb @ Wo1[...] + bo1[...])
    hb = jax.nn.relu(comb @ Wb1[...] + bb1[...])
    orange_ref[...] = (ho * Wo2[...].reshape(1, -1)).sum(
        axis=1, keepdims=True) + bo2[...]
    blue_ref[...] = (hb * Wb2[...].reshape(1, -1)).sum(
        axis=1, keepdims=True) + bb2[...]


def _lstm_heads(seq, gfeat, Wi1, Wh1, b1, Wi2, Wh2, b2,
                Wo1, bo1, Wo2, bo2, Wb1, bb1, Wb2, bb2):
    out_shape = (jax.ShapeDtypeStruct((BS, 1), jnp.float32),
                 jax.ShapeDtypeStruct((BS, 1), jnp.float32))
    return pl.pallas_call(
        _lstm_heads_body,
        out_shape=out_shape,
    )(seq, gfeat, Wi1, Wh1, b1, Wi2, Wh2, b2,
      Wo1, bo1, Wo2, bo2, Wb1, bb1, Wb2, bb2)


def _score_fold(We, ae, heads, fout):
    return jnp.einsum('khf,hf->kh', We.reshape(We.shape[0], heads, fout), ae)


def _pad_rows(a, m):
    return jnp.pad(a, ((0, m - a.shape[0]),) + ((0, 0),) * (a.ndim - 1))


def kernel(x, edge_index, edge_attr, batch, global_features, W1, as1, ad1, We1, ae1, b1, g1, be1, W2, as2, ad2, We2, ae2, b2, g2, be2, Wg, bg, Wi1, Wh1, bi1, bh1, Wi2, Wh2, bi2, bh2, Wo1, bo1, Wo2, bo2, Wb1, bb1, Wb2, bb2):
    src, dst = edge_index[0], edge_index[1]
    dst_p32 = jnp.concatenate(
        [dst, jnp.arange(E_PAD32 - E, dtype=jnp.int32) % N])

    WeS1 = _score_fold(We1, ae1, 4, HID)
    WeS2 = _score_fold(We2, ae2, 1, HID)
    es = edge_attr @ jnp.concatenate([WeS1, WeS2], axis=1)
    es1, es2 = es[:, :4], es[:, 4:5]

    h1t = _mm1(x, W1)
    hh1 = h1t.reshape(4, N, HID)
    asrc1 = x @ jnp.einsum('khf,hf->kh', W1.reshape(F_IN, 4, HID), as1)
    adst1 = x @ jnp.einsum('khf,hf->kh', W1.reshape(F_IN, 4, HID), ad1)

    scores1 = jnp.pad(jnp.concatenate([asrc1, adst1], axis=1),
                      ((0, DUPN - N), (0, 0)))
    ga1 = scores1[src][:, 0:4]
    gd1 = scores1[dst][:, 4:8]
    ee1 = jnp.exp(jax.nn.leaky_relu(ga1 + gd1 + es1, 0.2))
    pad16 = jnp.arange(E_PAD16 - E, dtype=jnp.int32) % N
    src_p16 = jnp.concatenate([src, pad16])
    dst_p16 = jnp.concatenate([dst, pad16])
    hoff = (jnp.arange(4, dtype=jnp.int32) * N)[:, None]
    gidx1 = (src_p16[None, :] + hoff).reshape(-1)

    rows1 = jnp.concatenate(
        [jnp.ones((E, 1), jnp.float32), es1, es2, ee1,
         jnp.zeros((E, 118), jnp.float32)], axis=1)
    rows1 = _pad_rows(rows1, E_PAD32)
    zeros128 = jnp.zeros((N_PAD // NS, HID), jnp.float32)
    st = _sc_segsum(rows1, dst_p32, zeros128, n_pad=N_PAD)
    st = (st[0] + st[1])[:N]
    deg, esum1, esum2, densum1 = st[:, 0], st[:, 1:5], st[:, 5:6], st[:, 6:10]
    invdeg = 1.0 / jnp.maximum(deg, 1.0)
    ls1 = esum1 * invdeg[:, None]
    ls2 = esum2 * invdeg[:, None]

    ef1 = jnp.exp(jax.nn.leaky_relu(asrc1 + adst1 + ls1, 0.2))
    denom1 = densum1 + ef1 + 1e-16
    alpha_f1 = ef1 / denom1
    rden1 = 1.0 / denom1

    ee1f = jnp.pad(ee1.T, ((0, 0), (0, E_PAD16 - E))).reshape(-1)
    agg1 = _sc_aggregate(h1t, gidx1, ee1f, dst_p16, zeros128, n_heads=4)
    agg1 = agg1.reshape(4, N_PAD, HID)[:, :N]
    out1 = agg1 * rden1.T[:, :, None] + hh1 * alpha_f1.T[:, :, None]
    y1 = out1.transpose(1, 0, 2).reshape(N, 4 * HID) + b1

    s1 = _col_stats(y1)
    mu1 = s1[0] / N
    var1 = s1[1] / N - mu1 * mu1
    sc1 = g1 / jnp.sqrt(var1 + 1e-5)
    sh1 = be1 - mu1 * sc1
    h2 = _bn_relu_mm(y1, sc1.reshape(1, -1), sh1.reshape(1, -1), W2)

    asrc2 = (h2 * as2).sum(1, keepdims=True)
    adst2 = (h2 * ad2).sum(1, keepdims=True)
    gidx2 = jnp.concatenate(
        [src, jnp.arange(E_PAD32 - E, dtype=jnp.int32) % N])
    scores2 = jnp.pad(jnp.concatenate(
        [asrc2, adst2, jnp.zeros((N, 6), jnp.float32)], axis=1),
        ((0, DUPN - N), (0, 0)))
    ga2 = scores2[src][:, 0:1]
    gd2 = scores2[dst][:, 1:2]
    ee2 = jnp.exp(jax.nn.leaky_relu(ga2 + gd2 + es2, 0.2))
    ee2f = jnp.pad(ee2[:, 0], (0, E_PAD32 - E))

    rows2 = jnp.concatenate(
        [ee2f[:, None], jnp.zeros((E_PAD32, 127), jnp.float32)], axis=1)
    d2 = _sc_segsum(rows2, dst_p32, zeros128, n_pad=N_PAD)
    densum2 = (d2[0] + d2[1])[:N, 0:1]
    ef2 = jnp.exp(jax.nn.leaky_relu(asrc2 + adst2 + ls2, 0.2))
    denom2 = densum2 + ef2 + 1e-16
    alpha_f2 = ef2 / denom2

    agg2 = _sc_aggregate(h2, gidx2, ee2f, dst_p32, zeros128, n_heads=1)
    agg2 = agg2.reshape(2, N_PAD, HID)
    y2 = (agg2[0, :N] + agg2[1, :N]) / denom2 + h2 * alpha_f2 + b2

    s2 = _col_stats(y2)
    mu2 = s2[0] / N
    var2 = s2[1] / N - mu2 * mu2
    sc2 = g2 / jnp.sqrt(var2 + 1e-5)
    sh2 = be2 - mu2 * sc2
    pvh, pvs = _bn_relu_gate_vals(y2, sc2.reshape(1, -1), sh2.reshape(1, -1),
                                  Wg.reshape(1, -1), bg.reshape(1, 1))
    pvh = _pad_rows(pvh, NP_PAD32)
    pvs = _pad_rows(pvs, NP_PAD32)
    batch_p = jnp.concatenate(
        [batch, jnp.arange(NP_PAD32 - N, dtype=jnp.int32) % NG])
    zpg = jnp.zeros((NG_PAD // NS, HID), jnp.float32)
    ph = _sc_segsum(pvh, batch_p, zpg, n_pad=NG_PAD)
    ps = _sc_segsum(pvs, batch_p, zpg, n_pad=NG_PAD)
    ph = (ph[0] + ph[1])[:NG]
    ps = (ps[0] + ps[1])[:NG, 0:1]
    graph_embeds = ph / (ps + 1e-16)

    seq = graph_embeds.reshape(BS, SEQ, HID).transpose(1, 0, 2)
    gfeat = global_features[SEQ - 1::SEQ]
    orange, blue = _lstm_heads(
        seq, gfeat, Wi1, Wh1, (bi1 + bh1).reshape(1, -1),
        Wi2, Wh2, (bi2 + bh2).reshape(1, -1),
        Wo1, bo1.reshape(1, -1), Wo2, bo2.reshape(1, 1),
        Wb1, bb1.reshape(1, -1), Wb2, bb2.reshape(1, 1))
    return (orange, blue)


# speedup vs baseline: 17.3157x; 1.0002x over previous
import functools
import jax
import jax.numpy as jnp
from jax import lax
from jax.experimental import pallas as pl
from jax.experimental.pallas import tpu as pltpu
from jax.experimental.pallas import tpu_sc as plsc

N = 10000
E = 160000
F_IN = 256
EDGE_DIM = 16
HID = 128
LSTM_H = 256
GFEAT = 32
NG = 240
SEQ = 6
BS = NG // SEQ

NC = 2
NS = 16
CH = 256

N_PAD = 10112
E_PAD16 = 163840
E_PAD32 = 163840
NP_PAD32 = 16384
NG_PAD = 256
DUPN = 320000


@functools.partial(jax.jit, static_argnames=("n_pad",))
def _sc_segsum(vals, idx, zeros, *, n_pad):
    M, W = vals.shape
    per_tile = M // (NC * NS)
    chunks = per_tile // CH
    rows_pt = n_pad // NS
    mesh = plsc.VectorSubcoreMesh(core_axis_name="c", subcore_axis_name="s")

    @functools.partial(
        pl.kernel, mesh=mesh,
        out_type=jax.ShapeDtypeStruct((NC * n_pad, W), jnp.float32),
        scratch_types=[
            pltpu.VMEM((CH,), jnp.int32),
            pltpu.VMEM((CH, W), jnp.float32),
            pltpu.VMEM_SHARED((n_pad, W), jnp.float32),
            pltpu.SemaphoreType.DMA,
        ],
    )
    def k(vals_hbm, idx_hbm, zeros_hbm, out_hbm, idx_v, val_v, acc_sh, sem):
        c = lax.axis_index("c")
        s = lax.axis_index("s")
        wid = s * NC + c

        if True:
            pltpu.sync_copy(zeros_hbm, acc_sh.at[pl.ds(s * rows_pt, rows_pt)])
            plsc.subcore_barrier()

            def body(i, carry):
                off = wid * per_tile + i * CH
                c1 = pltpu.async_copy(idx_hbm.at[pl.ds(off, CH)], idx_v, sem)
                c2 = pltpu.async_copy(vals_hbm.at[pl.ds(off, CH)], val_v, sem)
                c1.wait()
                c2.wait()
                pltpu.sync_copy(val_v, acc_sh.at[idx_v], add=True)
                return carry

            lax.fori_loop(0, chunks, body, 0)
            plsc.subcore_barrier()
            pltpu.sync_copy(acc_sh.at[pl.ds(s * rows_pt, rows_pt)],
                            out_hbm.at[pl.ds(c * n_pad + s * rows_pt, rows_pt)])

    return k(vals, idx, zeros).reshape(NC, n_pad, W)


def _sc_aggregate(table, gidx_flat, alpha_flat, dst, zeros, *, n_heads):
    if n_heads == 4:
        e_pad, heads_per_core, n_out_maj = E_PAD16, 2, 4
        per_tile = e_pad // NS
    else:
        e_pad, heads_per_core, n_out_maj = E_PAD32, 1, 2
        per_tile = e_pad // (NC * NS)
    chunks = per_tile // CH
    rows_pt = N_PAD // NS
    mesh = plsc.VectorSubcoreMesh(core_axis_name="c", subcore_axis_name="s")

    @functools.partial(
        pl.kernel, mesh=mesh,
        out_type=jax.ShapeDtypeStruct((n_out_maj * N_PAD, HID), jnp.float32),
        scratch_types=[
            pltpu.VMEM((CH,), jnp.int32),
            pltpu.VMEM((CH,), jnp.int32),
            pltpu.VMEM((CH,), jnp.float32),
            pltpu.VMEM((CH, HID), jnp.float32),
            pltpu.VMEM_SHARED((N_PAD, HID), jnp.float32),
            pltpu.SemaphoreType.DMA,
        ],
    )
    def k(table_hbm, gidx_hbm, alpha_hbm, dst_hbm, zeros_hbm, out_hbm,
          gidx_v, dst_v, alpha_v, rows_v, acc_sh, sem):
        c = lax.axis_index("c")
        s = lax.axis_index("s")

        if True:
            for klocal in range(heads_per_core):
                hd = c * heads_per_core + klocal
                pltpu.sync_copy(zeros_hbm,
                                acc_sh.at[pl.ds(s * rows_pt, rows_pt)])
                plsc.subcore_barrier()

                if n_heads == 4:
                    tile_base = s * per_tile
                else:
                    tile_base = (s * NC + c) * per_tile

                def body(i, carry):
                    off = tile_base + i * CH
                    goff = hd * e_pad + off if n_heads == 4 else off
                    c1 = pltpu.async_copy(gidx_hbm.at[pl.ds(goff, CH)],
                                          gidx_v, sem)
                    c2 = pltpu.async_copy(alpha_hbm.at[pl.ds(goff, CH)],
                                          alpha_v, sem)
                    c3 = pltpu.async_copy(dst_hbm.at[pl.ds(off, CH)],
                                          dst_v, sem)
                    c1.wait()
                    c2.wait()
                    c3.wait()
                    pltpu.async_copy(table_hbm.at[gidx_v], rows_v, sem).wait()

                    def scale(g, carry2):
                        av = alpha_v[pl.ds(g * 16, 16)]
                        for j16 in range(16):
                            a = jnp.full((16,), av[j16], jnp.float32)
                            row = g * 16 + j16
                            for l in range(HID // 16):
                                sl = pl.ds(l * 16, 16)
                                rows_v[row, sl] = rows_v[row, sl] * a
                        return carry2

                    lax.fori_loop(0, CH // 16, scale, 0)
                    pltpu.sync_copy(rows_v, acc_sh.at[dst_v], add=True)
                    return carry

                lax.fori_loop(0, chunks, body, 0)
                plsc.subcore_barrier()
                if n_heads == 4:
                    out_base = hd * N_PAD + s * rows_pt
                else:
                    out_base = c * N_PAD + s * rows_pt
                pltpu.sync_copy(acc_sh.at[pl.ds(s * rows_pt, rows_pt)],
                                out_hbm.at[pl.ds(out_base, rows_pt)])
                plsc.subcore_barrier()

    return k(table, gidx_flat, alpha_flat, dst, zeros)


def _mm1(x, W1):
    nb = 10
    bn = N // nb

    def body(x_ref, w_ref, o_ref):
        o_ref[...] = jnp.dot(x_ref[...], w_ref[...],
                             preferred_element_type=jnp.float32)

    return pl.pallas_call(
        body,
        grid=(nb, 4),
        in_specs=[
            pl.BlockSpec((bn, F_IN), lambda i, h: (i, 0)),
            pl.BlockSpec((F_IN, HID), lambda i, h: (0, h)),
        ],
        out_specs=pl.BlockSpec((bn, HID), lambda i, h: (h * nb + i, 0)),
        out_shape=jax.ShapeDtypeStruct((4 * N, HID), jnp.float32),
    )(x, W1)


def _col_stats(y):
    nb = 10
    bn = N // nb
    C = y.shape[1]

    def body(y_ref, o_ref):
        @pl.when(pl.program_id(0) == 0)
        def _():
            o_ref[...] = jnp.zeros((8, C), jnp.float32)
        blk = y_ref[...]
        o_ref[0:1, :] = o_ref[0:1, :] + blk.sum(axis=0, keepdims=True)
        o_ref[1:2, :] = o_ref[1:2, :] + (blk * blk).sum(axis=0, keepdims=True)

    return pl.pallas_call(
        body,
        grid=(nb,),
        in_specs=[pl.BlockSpec((bn, C), lambda i: (i, 0))],
        out_specs=pl.BlockSpec((8, C), lambda i: (0, 0)),
        out_shape=jax.ShapeDtypeStruct((8, C), jnp.float32),
    )(y)


def _bn_relu_mm(y, scale, shift, W):
    nb = 10
    bn = N // nb
    C = y.shape[1]
    Cout = W.shape[1]

    def body(y_ref, s_ref, b_ref, w_ref, o_ref):
        h = jax.nn.relu(y_ref[...] * s_ref[...] + b_ref[...])
        o_ref[...] = jnp.dot(h, w_ref[...], preferred_element_type=jnp.float32)

    return pl.pallas_call(
        body,
        grid=(nb,),
        in_specs=[
            pl.BlockSpec((bn, C), lambda i: (i, 0)),
            pl.BlockSpec((1, C), lambda i: (0, 0)),
            pl.BlockSpec((1, C), lambda i: (0, 0)),
            pl.BlockSpec((C, Cout), lambda i: (0, 0)),
        ],
        out_specs=pl.BlockSpec((bn, Cout), lambda i: (i, 0)),
        out_shape=jax.ShapeDtypeStruct((N, Cout), jnp.float32),
    )(y, scale, shift, W)


def _bn_relu_gate_vals(y2, scale, shift, Wg_row, bg):
    nb = 10
    bn = N // nb

    def body(y_ref, s_ref, b_ref, wg_ref, bg_ref, oh_ref, os_ref):
        hc = jax.nn.relu(y_ref[...] * s_ref[...] + b_ref[...])
        gl = (hc * wg_ref[...]).sum(axis=1, keepdims=True) + bg_ref[...]
        eg = jnp.exp(gl)
        oh_ref[...] = eg * hc
        os_ref[...] = jnp.concatenate(
            [eg, jnp.zeros((bn, 127), jnp.float32)], axis=1)

    return pl.pallas_call(
        body,
        grid=(nb,),
        in_specs=[
            pl.BlockSpec((bn, HID), lambda i: (i, 0)),
            pl.BlockSpec((1, HID), lambda i: (0, 0)),
            pl.BlockSpec((1, HID), lambda i: (0, 0)),
            pl.BlockSpec((1, HID), lambda i: (0, 0)),
            pl.BlockSpec((1, 1), lambda i: (0, 0)),
        ],
        out_specs=(pl.BlockSpec((bn, HID), lambda i: (i, 0)),
                   pl.BlockSpec((bn, HID), lambda i: (i, 0))),
        out_shape=(jax.ShapeDtypeStruct((N, HID), jnp.float32),
                   jax.ShapeDtypeStruct((N, HID), jnp.float32)),
    )(y2, scale, shift, Wg_row, bg)


def _lstm_heads_body(xseq, gfeat, Wi1, Wh1, b1, Wi2, Wh2, b2,
                     Wo1, bo1, Wo2, bo2, Wb1, bb1, Wb2, bb2,
                     orange_ref, blue_ref):
    def step(xt, carry):
        h1, c1, h2, c2 = carry
        g1 = xt @ Wi1[...] + h1 @ Wh1[...] + b1[...]
        i1 = jax.nn.sigmoid(g1[:, 0 * LSTM_H:1 * LSTM_H])
        f1 = jax.nn.sigmoid(g1[:, 1 * LSTM_H:2 * LSTM_H])
        gg1 = jnp.tanh(g1[:, 2 * LSTM_H:3 * LSTM_H])
        o1 = jax.nn.sigmoid(g1[:, 3 * LSTM_H:4 * LSTM_H])
        c1 = f1 * c1 + i1 * gg1
        h1 = o1 * jnp.tanh(c1)
        g2 = h1 @ Wi2[...] + h2 @ Wh2[...] + b2[...]
        i2 = jax.nn.sigmoid(g2[:, 0 * LSTM_H:1 * LSTM_H])
        f2 = jax.nn.sigmoid(g2[:, 1 * LSTM_H:2 * LSTM_H])
        gg2 = jnp.tanh(g2[:, 2 * LSTM_H:3 * LSTM_H])
        o2 = jax.nn.sigmoid(g2[:, 3 * LSTM_H:4 * LSTM_H])
        c2 = f2 * c2 + i2 * gg2
        h2 = o2 * jnp.tanh(c2)
        return h1, c1, h2, c2

    z = jnp.zeros((BS, LSTM_H), dtype=jnp.float32)
    carry = (z, z, z, z)
    for t in range(SEQ):
        carry = step(xseq[t], carry)
    h1, c1, h2, c2 = carry
    comb = jnp.concatenate([h2, gfeat[...]], axis=1)
    ho = jax.nn.relu(comb @ Wo1[...] + bo1[...])
    hb = jax.nn.relu(comb @ Wb1[...] + bb1[...])
    orange_ref[...] = (ho * Wo2[...].reshape(1, -1)).sum(
        axis=1, keepdims=True) + bo2[...]
    blue_ref[...] = (hb * Wb2[...].reshape(1, -1)).sum(
        axis=1, keepdims=True) + bb2[...]


def _lstm_heads(seq, gfeat, Wi1, Wh1, b1, Wi2, Wh2, b2,
                Wo1, bo1, Wo2, bo2, Wb1, bb1, Wb2, bb2):
    out_shape = (jax.ShapeDtypeStruct((BS, 1), jnp.float32),
                 jax.ShapeDtypeStruct((BS, 1), jnp.float32))
    return pl.pallas_call(
        _lstm_heads_body,
        out_shape=out_shape,
    )(seq, gfeat, Wi1, Wh1, b1, Wi2, Wh2, b2,
      Wo1, bo1, Wo2, bo2, Wb1, bb1, Wb2, bb2)


def _score_fold(We, ae, heads, fout):
    return jnp.einsum('khf,hf->kh', We.reshape(We.shape[0], heads, fout), ae)


def _pad_rows(a, m):
    return jnp.pad(a, ((0, m - a.shape[0]),) + ((0, 0),) * (a.ndim - 1))


def kernel(x, edge_index, edge_attr, batch, global_features, W1, as1, ad1, We1, ae1, b1, g1, be1, W2, as2, ad2, We2, ae2, b2, g2, be2, Wg, bg, Wi1, Wh1, bi1, bh1, Wi2, Wh2, bi2, bh2, Wo1, bo1, Wo2, bo2, Wb1, bb1, Wb2, bb2):
    src, dst = edge_index[0], edge_index[1]
    dst_p32 = jnp.concatenate(
        [dst, jnp.arange(E_PAD32 - E, dtype=jnp.int32) % N])

    WeS1 = _score_fold(We1, ae1, 4, HID)
    WeS2 = _score_fold(We2, ae2, 1, HID)
    es = edge_attr @ jnp.concatenate([WeS1, WeS2], axis=1)
    es1, es2 = es[:, :4], es[:, 4:5]

    h1t = _mm1(x, W1)
    hh1 = h1t.reshape(4, N, HID)
    asrc1 = x @ jnp.einsum('khf,hf->kh', W1.reshape(F_IN, 4, HID), as1)
    adst1 = x @ jnp.einsum('khf,hf->kh', W1.reshape(F_IN, 4, HID), ad1)

    scores1 = jnp.pad(jnp.concatenate([asrc1, adst1], axis=1),
                      ((0, DUPN - N), (0, 0)))
    ga1 = scores1[src][:, 0:4]
    gd1 = scores1[dst][:, 4:8]
    ee1 = jnp.exp(jax.nn.leaky_relu(ga1 + gd1 + es1, 0.2))
    pad16 = jnp.arange(E_PAD16 - E, dtype=jnp.int32) % N
    src_p16 = jnp.concatenate([src, pad16])
    dst_p16 = jnp.concatenate([dst, pad16])
    hoff = (jnp.arange(4, dtype=jnp.int32) * N)[:, None]
    gidx1 = (src_p16[None, :] + hoff).reshape(-1)

    rows1 = jnp.concatenate(
        [jnp.ones((E, 1), jnp.float32), es1, es2, ee1,
         jnp.zeros((E, 118), jnp.float32)], axis=1)
    rows1 = _pad_rows(rows1, E_PAD32)
    zeros128 = jnp.zeros((N_PAD // NS, HID), jnp.float32)
    st = _sc_segsum(rows1, dst_p32, zeros128, n_pad=N_PAD)
    st = (st[0] + st[1])[:N]
    deg, esum1, esum2, densum1 = st[:, 0], st[:, 1:5], st[:, 5:6], st[:, 6:10]
    invdeg = 1.0 / jnp.maximum(deg, 1.0)
    ls1 = esum1 * invdeg[:, None]
    ls2 = esum2 * invdeg[:, None]

    ef1 = jnp.exp(jax.nn.leaky_relu(asrc1 + adst1 + ls1, 0.2))
    denom1 = densum1 + ef1 + 1e-16
    alpha_f1 = ef1 / denom1
    rden1 = 1.0 / denom1

    ee1f = jnp.pad(ee1.T, ((0, 0), (0, E_PAD16 - E))).reshape(-1)
    agg1 = _sc_aggregate(h1t, gidx1, ee1f, dst_p16, zeros128, n_heads=4)
    agg1 = agg1.reshape(4, N_PAD, HID)[:, :N]
    out1 = agg1 * rden1.T[:, :, None] + hh1 * alpha_f1.T[:, :, None]
    y1 = out1.transpose(1, 0, 2).reshape(N, 4 * HID) + b1

    s1 = _col_stats(y1)
    mu1 = s1[0] / N
    var1 = s1[1] / N - mu1 * mu1
    sc1 = g1 / jnp.sqrt(var1 + 1e-5)
    sh1 = be1 - mu1 * sc1
    h2 = _bn_relu_mm(y1, sc1.reshape(1, -1), sh1.reshape(1, -1), W2)

    asrc2 = (h2 * as2).sum(1, keepdims=True)
    adst2 = (h2 * ad2).sum(1, keepdims=True)
    gidx2 = jnp.concatenate(
        [src, jnp.arange(E_PAD32 - E, dtype=jnp.int32) % N])
    scores2 = jnp.pad(jnp.concatenate(
        [asrc2, adst2, jnp.zeros((N, 6), jnp.float32)], axis=1),
        ((0, DUPN - N), (0, 0)))
    ga2 = scores2[src][:, 0:1]
    gd2 = scores2[dst][:, 1:2]
    ee2 = jnp.exp(jax.nn.leaky_relu(ga2 + gd2 + es2, 0.2))
    ee2f = jnp.pad(ee2[:, 0], (0, E_PAD32 - E))

    rows2 = jnp.concatenate(
        [ee2f[:, None], jnp.zeros((E_PAD32, 127), jnp.float32)], axis=1)
    d2 = _sc_segsum(rows2, dst_p32, zeros128, n_pad=N_PAD)
    densum2 = (d2[0] + d2[1])[:N, 0:1]
    ef2 = jnp.exp(jax.nn.leaky_relu(asrc2 + adst2 + ls2, 0.2))
    denom2 = densum2 + ef2 + 1e-16
    alpha_f2 = ef2 / denom2

    agg2 = _sc_aggregate(h2, gidx2, ee2f, dst_p32, zeros128, n_heads=1)
    agg2 = agg2.reshape(2, N_PAD, HID)
    y2 = (agg2[0, :N] + agg2[1, :N]) / denom2 + h2 * alpha_f2 + b2

    s2 = _col_stats(y2)
    mu2 = s2[0] / N
    var2 = s2[1] / N - mu2 * mu2
    sc2 = g2 / jnp.sqrt(var2 + 1e-5)
    sh2 = be2 - mu2 * sc2
    pvh, pvs = _bn_relu_gate_vals(y2, sc2.reshape(1, -1), sh2.reshape(1, -1),
                                  Wg.reshape(1, -1), bg.reshape(1, 1))
    pvh = _pad_rows(pvh, NP_PAD32)
    pvs = _pad_rows(pvs, NP_PAD32)
    batch_p = jnp.concatenate(
        [batch, jnp.arange(NP_PAD32 - N, dtype=jnp.int32) % NG])
    zpg = jnp.zeros((NG_PAD // NS, HID), jnp.float32)
    ph = _sc_segsum(pvh, batch_p, zpg, n_pad=NG_PAD)
    ps = _sc_segsum(pvs, batch_p, zpg, n_pad=NG_PAD)
    ph = (ph[0] + ph[1])[:NG]
    ps = (ps[0] + ps[1])[:NG, 0:1]
    graph_embeds = ph / (ps + 1e-16)

    seq = graph_embeds.reshape(BS, SEQ, HID).transpose(1, 0, 2)
    gfeat = global_features[SEQ - 1::SEQ]
    orange, blue = _lstm_heads(
        seq, gfeat, Wi1, Wh1, (bi1 + bh1).reshape(1, -1),
        Wi2, Wh2, (bi2 + bh2).reshape(1, -1),
        Wo1, bo1.reshape(1, -1), Wo2, bo2.reshape(1, 1),
        Wb1, bb1.reshape(1, -1), Wb2, bb2.reshape(1, 1))
    return (orange, blue)
